# baseline (speedup 1.0000x reference)
import functools

import jax
import jax.numpy as jnp
from jax import lax
from jax.experimental import pallas as pl
from jax.experimental.pallas import tpu as pltpu
from jax.experimental.pallas import tpu_sc as plsc

N, E, D = 10000, 160000, 256
K1, K2 = 8000, 6400
NPAD = 10240
DA = 272
B = 1024
GRID = NPAD // B


def _sage_body(a0, a1, a2, cnt_ref, x_ref, wl_ref, wr_ref,
               b_ref, p_ref, h_ref, s_ref):
    agg = a0[...] + a1[...] + a2[...]
    cnt = cnt_ref[...]
    mean = agg / jnp.maximum(cnt, 1.0)
    xv = x_ref[...]
    h = lax.dot_general(mean, wl_ref[...], (((1,), (1,)), ((), ())),
                        preferred_element_type=jnp.float32)
    h += lax.dot_general(xv, wr_ref[...], (((1,), (1,)), ((), ())),
                         preferred_element_type=jnp.float32)
    h = jnp.maximum(h + b_ref[...], 0.0)
    p = p_ref[...]
    pn = p * lax.rsqrt(jnp.sum(p * p))
    s = jnp.tanh(jnp.sum(h * pn, axis=1, keepdims=True))
    h_ref[...] = h
    s_ref[...] = s


def _sage(aggs, cnt, xin, wl, wr, b, p):
    return pl.pallas_call(
        _sage_body,
        grid=(GRID,),
        in_specs=[
            pl.BlockSpec((B, D), lambda i: (i, 0)),
            pl.BlockSpec((B, D), lambda i: (i, 0)),
            pl.BlockSpec((B, D), lambda i: (i, 0)),
            pl.BlockSpec((B, 1), lambda i: (i, 0)),
            pl.BlockSpec((B, D), lambda i: (i, 0)),
            pl.BlockSpec((D, D), lambda i: (0, 0)),
            pl.BlockSpec((D, D), lambda i: (0, 0)),
            pl.BlockSpec((1, D), lambda i: (0, 0)),
            pl.BlockSpec((1, D), lambda i: (0, 0)),
        ],
        out_specs=[
            pl.BlockSpec((B, D), lambda i: (i, 0)),
            pl.BlockSpec((B, 1), lambda i: (i, 0)),
        ],
        out_shape=[
            jax.ShapeDtypeStruct((NPAD, D), jnp.float32),
            jax.ShapeDtypeStruct((NPAD, 1), jnp.float32),
        ],
    )(*aggs, cnt, xin, wl, wr, b.reshape(1, D), p.reshape(1, D))


def _f32_sort_key(s):
    b = lax.bitcast_convert_type(s + 0.0, jnp.int32)
    k = jnp.where(b < 0, jnp.invert(b), b ^ jnp.int32(-2147483648))
    return k.astype(jnp.uint32)


def _thresh_body(k, use_keep, *refs):
    if use_keep:
        s_ref, keep_ref, out_ref = refs
    else:
        s_ref, out_ref = refs
        keep_ref = None
    key = _f32_sort_key(s_ref[...])
    rows = lax.broadcasted_iota(jnp.int32, (80, 128), 0)
    cols = lax.broadcasted_iota(jnp.int32, (80, 128), 1)
    valid = (rows * 128 + cols) < N
    if keep_ref is not None:
        valid = valid & (keep_ref[...] > 0.0)
    key = jnp.where(valid, key, jnp.uint32(0))
    prefix = jnp.uint32(0)
    for bit in range(31, -1, -1):
        cand = prefix | jnp.uint32(1 << bit)
        cnt = jnp.sum((key >= cand).astype(jnp.int32))
        prefix = jnp.where(cnt >= k, cand, prefix)
    out_ref[...] = ((key >= prefix) & valid).astype(jnp.float32)


def _threshold(score2d, k, keep2d=None):
    use_keep = keep2d is not None
    ins = (score2d, keep2d) if use_keep else (score2d,)
    return pl.pallas_call(
        functools.partial(_thresh_body, k, use_keep),
        out_shape=jax.ShapeDtypeStruct((80, 128), jnp.float32),
    )(*ins)


def _scale_body(h_ref, s_ref, k_ref, out_ref):
    out_ref[...] = h_ref[...] * (s_ref[...] * k_ref[...])


def _scale(h, s, keep):
    return pl.pallas_call(
        _scale_body,
        grid=(GRID,),
        in_specs=[
            pl.BlockSpec((B, D), lambda i: (i, 0)),
            pl.BlockSpec((B, 1), lambda i: (i, 0)),
            pl.BlockSpec((B, 1), lambda i: (i, 0)),
        ],
        out_specs=pl.BlockSpec((B, D), lambda i: (i, 0)),
        out_shape=jax.ShapeDtypeStruct((NPAD, D), jnp.float32),
    )(h, s, keep)


def _mean_body(h_ref, s_ref, k_ref, out_ref):
    @pl.when(pl.program_id(0) == 0)
    def _():
        out_ref[...] = jnp.zeros_like(out_ref)
    contrib = jnp.sum(h_ref[...] * (s_ref[...] * k_ref[...]),
                      axis=0, keepdims=True)
    out_ref[...] += contrib * (1.0 / K2)


def _final_mean(h, s, keep):
    return pl.pallas_call(
        _mean_body,
        grid=(GRID,),
        in_specs=[
            pl.BlockSpec((B, D), lambda i: (i, 0)),
            pl.BlockSpec((B, 1), lambda i: (i, 0)),
            pl.BlockSpec((B, 1), lambda i: (i, 0)),
        ],
        out_specs=pl.BlockSpec((1, D), lambda i: (0, 0)),
        out_shape=jax.ShapeDtypeStruct((1, D), jnp.float32),
    )(h, s, keep)


ECH = 512
EPAD = ((E + ECH - 1) // ECH) * ECH
EGRID = EPAD // ECH
NACC = NPAD + 8


NBANK = 3


def _tcseg_body(src_ref, dst_ref, x_ref, w_ref, *outs):
    aggs, cnt_ref, cnt_sm = outs[:NBANK], outs[NBANK], outs[NBANK + 1]

    @pl.when(pl.program_id(0) == 0)
    def _():
        for r in aggs:
            r[...] = jnp.zeros_like(r)

        def z(i, _):
            cnt_sm[i] = 0.0
            return 0
        lax.fori_loop(0, NACC, z, 0)

    def body(j, _):
        for k in range(NBANK):
            e = j * NBANK + k
            s = src_ref[0, 0, e]
            d = dst_ref[0, 0, e]
            aggs[k][pl.ds(d, 1), :] += x_ref[pl.ds(s, 1), :]
            cnt_sm[d] = cnt_sm[d] + w_ref[s]
        return 0
    lax.fori_loop(0, ECH // NBANK, body, 0, unroll=2)
    for e in range(ECH - ECH % NBANK, ECH):
        s = src_ref[0, 0, e]
        d = dst_ref[0, 0, e]
        aggs[0][pl.ds(d, 1), :] += x_ref[pl.ds(s, 1), :]
        cnt_sm[d] = cnt_sm[d] + w_ref[s]

    @pl.when(pl.program_id(0) == EGRID - 1)
    def _():
        def fin(i, _):
            cnt_ref[pl.ds(i, 1), :] = jnp.full((1, 1), cnt_sm[i],
                                               jnp.float32)
            return 0
        lax.fori_loop(0, NACC, fin, 0)


def _tcseg(xfeat, wcol, src3d, dst3d):
    return pl.pallas_call(
        _tcseg_body,
        grid=(EGRID,),
        in_specs=[
            pl.BlockSpec((1, 1, ECH), lambda i: (i, 0, 0),
                         memory_space=pltpu.SMEM),
            pl.BlockSpec((1, 1, ECH), lambda i: (i, 0, 0),
                         memory_space=pltpu.SMEM),
            pl.BlockSpec((NPAD, D), lambda i: (0, 0)),
            pl.BlockSpec((NPAD,), lambda i: (0,),
                         memory_space=pltpu.SMEM),
        ],
        out_specs=(
            [pl.BlockSpec((NACC, D), lambda i: (0, 0))] * NBANK
            + [pl.BlockSpec((NACC, 1), lambda i: (0, 0))]
        ),
        out_shape=(
            [jax.ShapeDtypeStruct((NACC, D), jnp.float32)] * NBANK
            + [jax.ShapeDtypeStruct((NACC, 1), jnp.float32)]
        ),
        scratch_shapes=[pltpu.SMEM((NACC,), jnp.float32)],
    )(src3d, dst3d, xfeat, wcol)


def kernel(x, edge_index, W1l, b1l, W1r, p1, W2l, b2l, W2r, p2):
    src = edge_index[0]
    dst = edge_index[1]
    src3d = jnp.pad(src, (0, EPAD - E)).reshape(EGRID, 1, ECH)
    dst3d = jnp.concatenate(
        [dst, jnp.full((EPAD - E,), NPAD, jnp.int32)]).reshape(EGRID, 1, ECH)
    x_p = jnp.pad(x, ((0, NPAD - N), (0, 0)))
    ones_col = jnp.ones((NPAD,), jnp.float32)

    ob1 = _tcseg(x_p, ones_col, src3d, dst3d)
    h1, s1 = _sage([a[:NPAD] for a in ob1[:NBANK]], ob1[NBANK][:NPAD],
                   x_p, W1l, W1r, b1l, p1)
    keep1 = _threshold(s1.reshape(80, 128), K1).reshape(NPAD, 1)
    g1 = _scale(h1, s1, keep1)

    ob2 = _tcseg(g1, keep1.reshape(NPAD), src3d, dst3d)
    h2, s2 = _sage([a[:NPAD] for a in ob2[:NBANK]], ob2[NBANK][:NPAD],
                   g1, W2l, W2r, b2l, p2)
    keep2 = _threshold(s2.reshape(80, 128), K2,
                       keep1.reshape(80, 128)).reshape(NPAD, 1)
    return _final_mean(h2, s2, keep2)


# speedup vs baseline: 1.2464x; 1.2464x over previous
import functools

import jax
import jax.numpy as jnp
from jax import lax
from jax.experimental import pallas as pl
from jax.experimental.pallas import tpu as pltpu
from jax.experimental.pallas import tpu_sc as plsc

N, E, D = 10000, 160000, 256
K1, K2 = 8000, 6400
NPAD = 10240
DA = 272
B = 1024
GRID = NPAD // B


def _sage_body(a0, a1, a2, c0, c1, x_ref, wl_ref, wr_ref,
               b_ref, p_ref, h_ref, s_ref):
    agg = a0[...] + a1[...] + a2[...]
    cnt = c0[...] + c1[...]
    mean = agg / jnp.maximum(cnt, 1.0)
    xv = x_ref[...]
    h = lax.dot_general(mean, wl_ref[...], (((1,), (1,)), ((), ())),
                        preferred_element_type=jnp.float32)
    h += lax.dot_general(xv, wr_ref[...], (((1,), (1,)), ((), ())),
                         preferred_element_type=jnp.float32)
    h = jnp.maximum(h + b_ref[...], 0.0)
    p = p_ref[...]
    pn = p * lax.rsqrt(jnp.sum(p * p))
    s = jnp.tanh(jnp.sum(h * pn, axis=1, keepdims=True))
    h_ref[...] = h
    s_ref[...] = s


def _sage(aggs, cnts, xin, wl, wr, b, p):
    return pl.pallas_call(
        _sage_body,
        grid=(GRID,),
        in_specs=[
            pl.BlockSpec((B, D), lambda i: (i, 0)),
            pl.BlockSpec((B, D), lambda i: (i, 0)),
            pl.BlockSpec((B, D), lambda i: (i, 0)),
            pl.BlockSpec((B, 1), lambda i: (i, 0)),
            pl.BlockSpec((B, 1), lambda i: (i, 0)),
            pl.BlockSpec((B, D), lambda i: (i, 0)),
            pl.BlockSpec((D, D), lambda i: (0, 0)),
            pl.BlockSpec((D, D), lambda i: (0, 0)),
            pl.BlockSpec((1, D), lambda i: (0, 0)),
            pl.BlockSpec((1, D), lambda i: (0, 0)),
        ],
        out_specs=[
            pl.BlockSpec((B, D), lambda i: (i, 0)),
            pl.BlockSpec((B, 1), lambda i: (i, 0)),
        ],
        out_shape=[
            jax.ShapeDtypeStruct((NPAD, D), jnp.float32),
            jax.ShapeDtypeStruct((NPAD, 1), jnp.float32),
        ],
    )(*aggs, *cnts, xin, wl, wr, b.reshape(1, D), p.reshape(1, D))


def _f32_sort_key(s):
    b = lax.bitcast_convert_type(s + 0.0, jnp.int32)
    k = jnp.where(b < 0, jnp.invert(b), b ^ jnp.int32(-2147483648))
    return k.astype(jnp.uint32)


def _thresh_body(k, use_keep, *refs):
    if use_keep:
        s_ref, keep_ref, out_ref = refs
    else:
        s_ref, out_ref = refs
        keep_ref = None
    key = _f32_sort_key(s_ref[...])
    rows = lax.broadcasted_iota(jnp.int32, (80, 128), 0)
    cols = lax.broadcasted_iota(jnp.int32, (80, 128), 1)
    valid = (rows * 128 + cols) < N
    if keep_ref is not None:
        valid = valid & (keep_ref[...] > 0.0)
    key = jnp.where(valid, key, jnp.uint32(0))
    prefix = jnp.uint32(0)
    for bit in range(31, -1, -1):
        cand = prefix | jnp.uint32(1 << bit)
        cnt = jnp.sum((key >= cand).astype(jnp.int32))
        prefix = jnp.where(cnt >= k, cand, prefix)
    out_ref[...] = ((key >= prefix) & valid).astype(jnp.float32)


def _threshold(score2d, k, keep2d=None):
    use_keep = keep2d is not None
    ins = (score2d, keep2d) if use_keep else (score2d,)
    return pl.pallas_call(
        functools.partial(_thresh_body, k, use_keep),
        out_shape=jax.ShapeDtypeStruct((80, 128), jnp.float32),
    )(*ins)


def _scale_body(h_ref, s_ref, k_ref, out_ref):
    out_ref[...] = h_ref[...] * (s_ref[...] * k_ref[...])


def _scale(h, s, keep):
    return pl.pallas_call(
        _scale_body,
        grid=(GRID,),
        in_specs=[
            pl.BlockSpec((B, D), lambda i: (i, 0)),
            pl.BlockSpec((B, 1), lambda i: (i, 0)),
            pl.BlockSpec((B, 1), lambda i: (i, 0)),
        ],
        out_specs=pl.BlockSpec((B, D), lambda i: (i, 0)),
        out_shape=jax.ShapeDtypeStruct((NPAD, D), jnp.float32),
    )(h, s, keep)


def _mean_body(h_ref, s_ref, k_ref, out_ref):
    @pl.when(pl.program_id(0) == 0)
    def _():
        out_ref[...] = jnp.zeros_like(out_ref)
    contrib = jnp.sum(h_ref[...] * (s_ref[...] * k_ref[...]),
                      axis=0, keepdims=True)
    out_ref[...] += contrib * (1.0 / K2)


def _final_mean(h, s, keep):
    return pl.pallas_call(
        _mean_body,
        grid=(GRID,),
        in_specs=[
            pl.BlockSpec((B, D), lambda i: (i, 0)),
            pl.BlockSpec((B, 1), lambda i: (i, 0)),
            pl.BlockSpec((B, 1), lambda i: (i, 0)),
        ],
        out_specs=pl.BlockSpec((1, D), lambda i: (0, 0)),
        out_shape=jax.ShapeDtypeStruct((1, D), jnp.float32),
    )(h, s, keep)


ECH = 512
EPAD = ((E + ECH - 1) // ECH) * ECH
EGRID = EPAD // ECH
NACC = NPAD + 8


NBANK = 3
CBANK = 2


def _tcseg_body(src_ref, dst_ref, x_ref, w_ref, *outs):
    aggs, cnts = outs[:NBANK], outs[NBANK:]

    @pl.when(pl.program_id(0) == 0)
    def _():
        for r in aggs + cnts:
            r[...] = jnp.zeros_like(r)

    def _edge(e, k):
        s = src_ref[0, 0, e]
        d = dst_ref[0, 0, e]
        aggs[k % NBANK][pl.ds(d, 1), :] += x_ref[pl.ds(s, 1), :]
        cnts[k % CBANK][pl.ds(d, 1), :] += w_ref[pl.ds(s, 1), :]

    def body(j, _):
        for k in range(NBANK * CBANK):
            _edge(j * (NBANK * CBANK) + k, k)
        return 0
    lax.fori_loop(0, ECH // (NBANK * CBANK), body, 0, unroll=1)
    for i, e in enumerate(range(ECH - ECH % (NBANK * CBANK), ECH)):
        _edge(e, i)


def _tcseg(xfeat, wcol, src3d, dst3d):
    return pl.pallas_call(
        _tcseg_body,
        grid=(EGRID,),
        in_specs=[
            pl.BlockSpec((1, 1, ECH), lambda i: (i, 0, 0),
                         memory_space=pltpu.SMEM),
            pl.BlockSpec((1, 1, ECH), lambda i: (i, 0, 0),
                         memory_space=pltpu.SMEM),
            pl.BlockSpec((NPAD, D), lambda i: (0, 0)),
            pl.BlockSpec((NPAD, 1), lambda i: (0, 0)),
        ],
        out_specs=(
            [pl.BlockSpec((NACC, D), lambda i: (0, 0))] * NBANK
            + [pl.BlockSpec((NACC, 1), lambda i: (0, 0))] * CBANK
        ),
        out_shape=(
            [jax.ShapeDtypeStruct((NACC, D), jnp.float32)] * NBANK
            + [jax.ShapeDtypeStruct((NACC, 1), jnp.float32)] * CBANK
        ),
    )(src3d, dst3d, xfeat, wcol)


def kernel(x, edge_index, W1l, b1l, W1r, p1, W2l, b2l, W2r, p2):
    src = edge_index[0]
    dst = edge_index[1]
    src3d = jnp.pad(src, (0, EPAD - E)).reshape(EGRID, 1, ECH)
    dst3d = jnp.concatenate(
        [dst, jnp.full((EPAD - E,), NPAD, jnp.int32)]).reshape(EGRID, 1, ECH)
    x_p = jnp.pad(x, ((0, NPAD - N), (0, 0)))
    ones_col = jnp.ones((NPAD, 1), jnp.float32)

    ob1 = _tcseg(x_p, ones_col, src3d, dst3d)
    h1, s1 = _sage([a[:NPAD] for a in ob1[:NBANK]],
                   [c[:NPAD] for c in ob1[NBANK:]], x_p, W1l, W1r, b1l, p1)
    keep1 = _threshold(s1.reshape(80, 128), K1).reshape(NPAD, 1)
    g1 = _scale(h1, s1, keep1)

    ob2 = _tcseg(g1, keep1, src3d, dst3d)
    h2, s2 = _sage([a[:NPAD] for a in ob2[:NBANK]],
                   [c[:NPAD] for c in ob2[NBANK:]], g1, W2l, W2r, b2l, p2)
    keep2 = _threshold(s2.reshape(80, 128), K2,
                       keep1.reshape(80, 128)).reshape(NPAD, 1)
    return _final_mean(h2, s2, keep2)


# speedup vs baseline: 1.3987x; 1.1222x over previous
import functools

import jax
import jax.numpy as jnp
from jax import lax
from jax.experimental import pallas as pl
from jax.experimental.pallas import tpu as pltpu
from jax.experimental.pallas import tpu_sc as plsc

N, E, D = 10000, 160000, 256
K1, K2 = 8000, 6400
NPAD = 10240
DA = 272
B = 1024
GRID = NPAD // B


def _sage_body(a0, a1, a2, c0, c1, x_ref, wl_ref, wr_ref,
               b_ref, p_ref, h_ref, s_ref):
    agg = a0[...] + a1[...] + a2[...]
    cnt = c0[...] + c1[...]
    mean = agg / jnp.maximum(cnt, 1.0)
    xv = x_ref[...]
    h = lax.dot_general(mean, wl_ref[...], (((1,), (1,)), ((), ())),
                        preferred_element_type=jnp.float32)
    h += lax.dot_general(xv, wr_ref[...], (((1,), (1,)), ((), ())),
                         preferred_element_type=jnp.float32)
    h = jnp.maximum(h + b_ref[...], 0.0)
    p = p_ref[...]
    pn = p * lax.rsqrt(jnp.sum(p * p))
    s = jnp.tanh(jnp.sum(h * pn, axis=1, keepdims=True))
    h_ref[...] = h
    s_ref[...] = s


def _sage(aggs, cnts, xin, wl, wr, b, p):
    return pl.pallas_call(
        _sage_body,
        grid=(GRID,),
        in_specs=[
            pl.BlockSpec((B, D), lambda i: (i, 0)),
            pl.BlockSpec((B, D), lambda i: (i, 0)),
            pl.BlockSpec((B, D), lambda i: (i, 0)),
            pl.BlockSpec((B, 1), lambda i: (i, 0)),
            pl.BlockSpec((B, 1), lambda i: (i, 0)),
            pl.BlockSpec((B, D), lambda i: (i, 0)),
            pl.BlockSpec((D, D), lambda i: (0, 0)),
            pl.BlockSpec((D, D), lambda i: (0, 0)),
            pl.BlockSpec((1, D), lambda i: (0, 0)),
            pl.BlockSpec((1, D), lambda i: (0, 0)),
        ],
        out_specs=[
            pl.BlockSpec((B, D), lambda i: (i, 0)),
            pl.BlockSpec((B, 1), lambda i: (i, 0)),
        ],
        out_shape=[
            jax.ShapeDtypeStruct((NPAD, D), jnp.float32),
            jax.ShapeDtypeStruct((NPAD, 1), jnp.float32),
        ],
    )(*aggs, *cnts, xin, wl, wr, b.reshape(1, D), p.reshape(1, D))


def _f32_sort_key(s):
    b = lax.bitcast_convert_type(s + 0.0, jnp.int32)
    k = jnp.where(b < 0, jnp.invert(b), b ^ jnp.int32(-2147483648))
    return k.astype(jnp.uint32)


def _thresh_body(k, use_keep, *refs):
    if use_keep:
        s_ref, keep_ref, out_ref = refs
    else:
        s_ref, out_ref = refs
        keep_ref = None
    key = _f32_sort_key(s_ref[...])
    rows = lax.broadcasted_iota(jnp.int32, (80, 128), 0)
    cols = lax.broadcasted_iota(jnp.int32, (80, 128), 1)
    valid = (rows * 128 + cols) < N
    if keep_ref is not None:
        valid = valid & (keep_ref[...] > 0.0)
    key = jnp.where(valid, key, jnp.uint32(0))
    prefix = jnp.uint32(0)
    for bit in range(31, -1, -1):
        cand = prefix | jnp.uint32(1 << bit)
        cnt = jnp.sum((key >= cand).astype(jnp.int32))
        prefix = jnp.where(cnt >= k, cand, prefix)
    out_ref[...] = ((key >= prefix) & valid).astype(jnp.float32)


def _threshold(score2d, k, keep2d=None):
    use_keep = keep2d is not None
    ins = (score2d, keep2d) if use_keep else (score2d,)
    return pl.pallas_call(
        functools.partial(_thresh_body, k, use_keep),
        out_shape=jax.ShapeDtypeStruct((80, 128), jnp.float32),
    )(*ins)


def _scale_body(h_ref, s_ref, k_ref, out_ref):
    out_ref[...] = h_ref[...] * (s_ref[...] * k_ref[...])


def _scale(h, s, keep):
    return pl.pallas_call(
        _scale_body,
        grid=(GRID,),
        in_specs=[
            pl.BlockSpec((B, D), lambda i: (i, 0)),
            pl.BlockSpec((B, 1), lambda i: (i, 0)),
            pl.BlockSpec((B, 1), lambda i: (i, 0)),
        ],
        out_specs=pl.BlockSpec((B, D), lambda i: (i, 0)),
        out_shape=jax.ShapeDtypeStruct((NPAD, D), jnp.float32),
    )(h, s, keep)


def _mean_body(h_ref, s_ref, k_ref, out_ref):
    @pl.when(pl.program_id(0) == 0)
    def _():
        out_ref[...] = jnp.zeros_like(out_ref)
    contrib = jnp.sum(h_ref[...] * (s_ref[...] * k_ref[...]),
                      axis=0, keepdims=True)
    out_ref[...] += contrib * (1.0 / K2)


def _final_mean(h, s, keep):
    return pl.pallas_call(
        _mean_body,
        grid=(GRID,),
        in_specs=[
            pl.BlockSpec((B, D), lambda i: (i, 0)),
            pl.BlockSpec((B, 1), lambda i: (i, 0)),
            pl.BlockSpec((B, 1), lambda i: (i, 0)),
        ],
        out_specs=pl.BlockSpec((1, D), lambda i: (0, 0)),
        out_shape=jax.ShapeDtypeStruct((1, D), jnp.float32),
    )(h, s, keep)


ECH = 512
EPAD = ((E + ECH - 1) // ECH) * ECH
EGRID = EPAD // ECH
NACC = NPAD + 8


NBANK = 3
CBANK = 2


def _tcseg_body(src_ref, dst_ref, x_ref, w_ref, *outs):
    aggs, cnts = outs[:NBANK], outs[NBANK:]

    @pl.when(pl.program_id(0) == 0)
    def _():
        for r in aggs + cnts:
            r[...] = jnp.zeros_like(r)

    def _edge(e, k):
        s = src_ref[0, 0, e]
        d = dst_ref[0, 0, e]
        aggs[k % NBANK][pl.ds(d, 1), :] += x_ref[pl.ds(s, 1), :]
        cnts[k % CBANK][pl.ds(d, 1), :] += w_ref[pl.ds(s, 1), :]

    def body(j, _):
        for k in range(NBANK * CBANK):
            _edge(j * (NBANK * CBANK) + k, k)
        return 0
    lax.fori_loop(0, ECH // (NBANK * CBANK), body, 0, unroll=2)
    for i, e in enumerate(range(ECH - ECH % (NBANK * CBANK), ECH)):
        _edge(e, i)


def _tcseg(xfeat, wcol, src3d, dst3d):
    return pl.pallas_call(
        _tcseg_body,
        grid=(EGRID,),
        in_specs=[
            pl.BlockSpec((1, 1, ECH), lambda i: (i, 0, 0),
                         memory_space=pltpu.SMEM),
            pl.BlockSpec((1, 1, ECH), lambda i: (i, 0, 0),
                         memory_space=pltpu.SMEM),
            pl.BlockSpec((NPAD, D), lambda i: (0, 0)),
            pl.BlockSpec((NPAD, 1), lambda i: (0, 0)),
        ],
        out_specs=(
            [pl.BlockSpec((NACC, D), lambda i: (0, 0))] * NBANK
            + [pl.BlockSpec((NACC, 1), lambda i: (0, 0))] * CBANK
        ),
        out_shape=(
            [jax.ShapeDtypeStruct((NACC, D), jnp.float32)] * NBANK
            + [jax.ShapeDtypeStruct((NACC, 1), jnp.float32)] * CBANK
        ),
    )(src3d, dst3d, xfeat, wcol)


def kernel(x, edge_index, W1l, b1l, W1r, p1, W2l, b2l, W2r, p2):
    src = edge_index[0]
    dst = edge_index[1]
    src3d = jnp.pad(src, (0, EPAD - E)).reshape(EGRID, 1, ECH)
    dst3d = jnp.concatenate(
        [dst, jnp.full((EPAD - E,), NPAD, jnp.int32)]).reshape(EGRID, 1, ECH)
    x_p = jnp.pad(x, ((0, NPAD - N), (0, 0)))
    ones_col = jnp.ones((NPAD, 1), jnp.float32)

    ob1 = _tcseg(x_p, ones_col, src3d, dst3d)
    h1, s1 = _sage([a[:NPAD] for a in ob1[:NBANK]],
                   [c[:NPAD] for c in ob1[NBANK:]], x_p, W1l, W1r, b1l, p1)
    keep1 = _threshold(s1.reshape(80, 128), K1).reshape(NPAD, 1)
    g1 = _scale(h1, s1, keep1)

    ob2 = _tcseg(g1, keep1, src3d, dst3d)
    h2, s2 = _sage([a[:NPAD] for a in ob2[:NBANK]],
                   [c[:NPAD] for c in ob2[NBANK:]], g1, W2l, W2r, b2l, p2)
    keep2 = _threshold(s2.reshape(80, 128), K2,
                       keep1.reshape(80, 128)).reshape(NPAD, 1)
    return _final_mean(h2, s2, keep2)
